# baseline (speedup 1.0000x reference)
import functools

import jax
import jax.numpy as jnp
from jax import lax
from jax.experimental import pallas as pl
from jax.experimental.pallas import tpu as pltpu

_S = 512
_D = 512
_MAX_REL = 32
_VOCAB = 2 * _MAX_REL + 1
_KPAD = 128


_SB = 128


def _rpe_kernel(x_ref, pe_ref, out_ref, rowpe_ref):
    sb = pl.program_id(0)
    b = pl.program_id(1)

    @pl.when(b == 0)
    def _compute_row_pe():
        base = sb * _SB
        i = base + lax.broadcasted_iota(jnp.int32, (_SB, _KPAD), 0)
        k = lax.broadcasted_iota(jnp.int32, (_SB, _KPAD), 1)
        r = k - _MAX_REL
        mid = ((k >= 1) & (k <= _VOCAB - 2) & (r <= i) & (r >= i - (_S - 1)))
        counts = mid.astype(jnp.float32)
        left = jnp.maximum(_S - _MAX_REL - i, 0).astype(jnp.float32)
        right = jnp.maximum(i - _MAX_REL + 1, 0).astype(jnp.float32)
        counts = counts + jnp.where(k == 0, left, 0.0)
        counts = counts + jnp.where(k == _VOCAB - 1, right, 0.0)
        rowpe_ref[...] = jnp.dot(
            counts, pe_ref[...], preferred_element_type=jnp.float32
        ) * (1.0 / _S)

    out_ref[0] = x_ref[0] + rowpe_ref[...]


@jax.jit
def kernel(x, pe):
    b, s, d = x.shape
    pe_padded = jnp.zeros((_KPAD, d), dtype=pe.dtype).at[: pe.shape[0]].set(pe)
    return pl.pallas_call(
        _rpe_kernel,
        grid=(s // _SB, b),
        in_specs=[
            pl.BlockSpec((1, _SB, d), lambda sb, i: (i, sb, 0)),
            pl.BlockSpec((_KPAD, d), lambda sb, i: (0, 0)),
        ],
        out_specs=pl.BlockSpec((1, _SB, d), lambda sb, i: (i, sb, 0)),
        out_shape=jax.ShapeDtypeStruct((b, s, d), x.dtype),
        scratch_shapes=[pltpu.VMEM((_SB, d), jnp.float32)],
        compiler_params=pltpu.CompilerParams(
            dimension_semantics=("arbitrary", "arbitrary"),
        ),
    )(x, pe_padded)


# speedup vs baseline: 2.4345x; 2.4345x over previous
import functools

import jax
import jax.numpy as jnp
from jax import lax
from jax.experimental import pallas as pl
from jax.experimental.pallas import tpu as pltpu

_S = 512
_D = 512
_MAX_REL = 32
_VOCAB = 2 * _MAX_REL + 1
_KPAD = 128
_BB = 2


def _rpe_kernel(x_ref, pe_ref, out_ref, rowpe_ref):
    b = pl.program_id(0)

    @pl.when(b == 0)
    def _compute_row_pe():
        i = lax.broadcasted_iota(jnp.int32, (_S, _KPAD), 0)
        k = lax.broadcasted_iota(jnp.int32, (_S, _KPAD), 1)
        r = k - _MAX_REL
        mid = ((k >= 1) & (k <= _VOCAB - 2) & (r <= i) & (r >= i - (_S - 1)))
        counts = mid.astype(jnp.float32)
        left = jnp.maximum(_S - _MAX_REL - i, 0).astype(jnp.float32)
        right = jnp.maximum(i - _MAX_REL + 1, 0).astype(jnp.float32)
        counts = counts + jnp.where(k == 0, left, 0.0)
        counts = counts + jnp.where(k == _VOCAB - 1, right, 0.0)
        rowpe_ref[...] = jnp.dot(
            counts, pe_ref[...], preferred_element_type=jnp.float32
        ) * (1.0 / _S)

    out_ref[...] = x_ref[...] + rowpe_ref[...][None]


@jax.jit
def kernel(x, pe):
    b, s, d = x.shape
    pe_padded = jnp.zeros((_KPAD, d), dtype=pe.dtype).at[: pe.shape[0]].set(pe)
    return pl.pallas_call(
        _rpe_kernel,
        grid=(b // _BB,),
        in_specs=[
            pl.BlockSpec((_BB, s, d), lambda i: (i, 0, 0)),
            pl.BlockSpec((_KPAD, d), lambda i: (0, 0)),
        ],
        out_specs=pl.BlockSpec((_BB, s, d), lambda i: (i, 0, 0)),
        out_shape=jax.ShapeDtypeStruct((b, s, d), x.dtype),
        scratch_shapes=[pltpu.VMEM((s, d), jnp.float32)],
        compiler_params=pltpu.CompilerParams(
            dimension_semantics=("arbitrary",),
        ),
    )(x, pe_padded)


# speedup vs baseline: 2.8473x; 1.1695x over previous
import functools

import jax
import jax.numpy as jnp
from jax import lax
from jax.experimental import pallas as pl
from jax.experimental.pallas import tpu as pltpu

_S = 512
_D = 512
_MAX_REL = 32
_VOCAB = 2 * _MAX_REL + 1
_KPAD = 128
_BB = 4


def _rpe_kernel(x_ref, pe_ref, out_ref, rowpe_ref):
    b = pl.program_id(0)

    @pl.when(b == 0)
    def _compute_row_pe():
        i = lax.broadcasted_iota(jnp.int32, (_S, _KPAD), 0)
        k = lax.broadcasted_iota(jnp.int32, (_S, _KPAD), 1)
        r = k - _MAX_REL
        mid = ((k >= 1) & (k <= _VOCAB - 2) & (r <= i) & (r >= i - (_S - 1)))
        counts = mid.astype(jnp.float32)
        left = jnp.maximum(_S - _MAX_REL - i, 0).astype(jnp.float32)
        right = jnp.maximum(i - _MAX_REL + 1, 0).astype(jnp.float32)
        counts = counts + jnp.where(k == 0, left, 0.0)
        counts = counts + jnp.where(k == _VOCAB - 1, right, 0.0)
        rowpe_ref[...] = jnp.dot(
            counts, pe_ref[...], preferred_element_type=jnp.float32
        ) * (1.0 / _S)

    out_ref[...] = x_ref[...] + rowpe_ref[...][None]


@jax.jit
def kernel(x, pe):
    b, s, d = x.shape
    pe_padded = jnp.zeros((_KPAD, d), dtype=pe.dtype).at[: pe.shape[0]].set(pe)
    return pl.pallas_call(
        _rpe_kernel,
        grid=(b // _BB,),
        in_specs=[
            pl.BlockSpec((_BB, s, d), lambda i: (i, 0, 0)),
            pl.BlockSpec((_KPAD, d), lambda i: (0, 0)),
        ],
        out_specs=pl.BlockSpec((_BB, s, d), lambda i: (i, 0, 0)),
        out_shape=jax.ShapeDtypeStruct((b, s, d), x.dtype),
        scratch_shapes=[pltpu.VMEM((s, d), jnp.float32)],
        compiler_params=pltpu.CompilerParams(
            dimension_semantics=("arbitrary",),
        ),
    )(x, pe_padded)
